# Initial kernel scaffold; baseline (speedup 1.0000x reference)
import jax
import jax.numpy as jnp
from jax.experimental import pallas as pl


def kernel(x, embed):
    raise NotImplementedError("write your pallas kernel here")


# speedup vs baseline: 1.4590x; 1.4590x over previous
import functools

import jax
import jax.numpy as jnp
from jax import lax
from jax.experimental import pallas as pl
from jax.experimental.pallas import tpu as pltpu
from jax.experimental.pallas import tpu_sc as plsc

_B = 4096
_H = 200
_D = 32
_BT = _B * _H
_NW = 32
_PER_W = _BT // _NW
_CHUNK = 1024
_NIDX = _CHUNK // 128
_NSTEP = _PER_W // _CHUNK


def _body(x_hbm, tab_hbm, out_hbm, idx_v, rows_v, sem):
    wid = lax.axis_index("s") * 2 + lax.axis_index("c")
    base = wid * _PER_W

    def step(c, _):
        off = pl.multiple_of(base + c * _CHUNK, _CHUNK)
        pltpu.sync_copy(x_hbm.at[pl.ds(pl.multiple_of(off // 128, 8), _NIDX)], idx_v)
        copies = [
            pltpu.async_copy(
                tab_hbm.at[idx_v.at[j]],
                rows_v.at[pl.ds(j * 128, 128)],
                sem,
            )
            for j in range(_NIDX)
        ]
        for cp in copies:
            cp.wait()
        pltpu.sync_copy(rows_v, out_hbm.at[pl.ds(off, _CHUNK)])
        return ()

    lax.fori_loop(0, _NSTEP, step, ())


@jax.jit
def kernel(x, embed):
    x2d = x.reshape(_BT // 128, 128)
    run = functools.partial(
        pl.kernel,
        mesh=plsc.VectorSubcoreMesh(core_axis_name="c", subcore_axis_name="s"),
        out_type=jax.ShapeDtypeStruct((_BT, _D), jnp.float32),
        scratch_types=[
            pltpu.VMEM((_NIDX, 128), jnp.int32),
            pltpu.VMEM((_CHUNK, _D), jnp.float32),
            pltpu.SemaphoreType.DMA,
        ],
        compiler_params=pltpu.CompilerParams(use_tc_tiling_on_sc=False),
    )(_body)
    out = run(x2d, embed)
    return out.reshape(_B, _H, _D)


# speedup vs baseline: 1.4852x; 1.0179x over previous
import functools

import jax
import jax.numpy as jnp
from jax import lax
from jax.experimental import pallas as pl
from jax.experimental.pallas import tpu as pltpu
from jax.experimental.pallas import tpu_sc as plsc

_B = 4096
_H = 200
_D = 32
_BT = _B * _H
_NW = 32
_PER_W = _BT // _NW
_CHUNK = 1024
_NIDX = _CHUNK // 128
_NSTEP = _PER_W // _CHUNK


def _body(x_hbm, tab_hbm, out_hbm, idx_v, rows_v, gsem0, gsem1, wsem0, wsem1):
    wid = lax.axis_index("s") * 2 + lax.axis_index("c")
    base = wid * _PER_W
    gsem = (gsem0, gsem1)
    wsem = (wsem0, wsem1)

    def _off(c):
        return pl.multiple_of(base + c * _CHUNK, _CHUNK)

    def gather(c, b):
        off = _off(c)
        pltpu.sync_copy(
            x_hbm.at[pl.ds(pl.multiple_of(off // 128, 8), _NIDX)], idx_v.at[b]
        )
        for j in range(_NIDX):
            pltpu.async_copy(
                tab_hbm.at[idx_v.at[b, j]],
                rows_v.at[b, pl.ds(j * 128, 128)],
                gsem[b],
            )

    def drain_gather(b):
        pltpu.make_async_copy(
            out_hbm.at[pl.ds(0, _CHUNK)], rows_v.at[b], gsem[b]
        ).wait()

    def writeout(c, b):
        pltpu.async_copy(rows_v.at[b], out_hbm.at[pl.ds(_off(c), _CHUNK)], wsem[b])

    def drain_writeout(b):
        pltpu.make_async_copy(
            rows_v.at[b], out_hbm.at[pl.ds(0, _CHUNK)], wsem[b]
        ).wait()

    gather(0, 0)
    drain_gather(0)
    writeout(0, 0)
    gather(1, 1)

    @pl.loop(1, (_NSTEP - 1) // 2)
    def _(g):
        c = pl.multiple_of(2 * g, 2)
        drain_gather(1)
        writeout(c - 1, 1)
        drain_writeout(0)
        gather(c, 0)
        drain_gather(0)
        writeout(c, 0)
        drain_writeout(1)
        gather(c + 1, 1)

    drain_gather(1)
    writeout(_NSTEP - 2, 1)
    drain_writeout(0)
    gather(_NSTEP - 1, 0)
    drain_gather(0)
    writeout(_NSTEP - 1, 0)
    drain_writeout(1)
    drain_writeout(0)


@jax.jit
def kernel(x, embed):
    x2d = x.reshape(_BT // 128, 128)
    run = functools.partial(
        pl.kernel,
        mesh=plsc.VectorSubcoreMesh(core_axis_name="c", subcore_axis_name="s"),
        out_type=jax.ShapeDtypeStruct((_BT, _D), jnp.float32),
        scratch_types=[
            pltpu.VMEM((2, _NIDX, 128), jnp.int32),
            pltpu.VMEM((2, _CHUNK, _D), jnp.float32),
            pltpu.SemaphoreType.DMA,
            pltpu.SemaphoreType.DMA,
            pltpu.SemaphoreType.DMA,
            pltpu.SemaphoreType.DMA,
        ],
        compiler_params=pltpu.CompilerParams(use_tc_tiling_on_sc=False),
    )(_body)
    out = run(x2d, embed)
    return out.reshape(_B, _H, _D)


# speedup vs baseline: 1.4906x; 1.0036x over previous
import functools

import jax
import jax.numpy as jnp
from jax import lax
from jax.experimental import pallas as pl
from jax.experimental.pallas import tpu as pltpu
from jax.experimental.pallas import tpu_sc as plsc

_B = 4096
_H = 200
_D = 32
_NW = 32
_ROWS_W = _B // _NW
_CB = 8
_NSTEP = _ROWS_W // _CB


def _body(x_hbm, tab_hbm, out_hbm, idx_v, rows_v, gsem0, gsem1, wsem0, wsem1):
    wid = lax.axis_index("s") * 2 + lax.axis_index("c")
    base = wid * _ROWS_W
    gsem = (gsem0, gsem1)
    wsem = (wsem0, wsem1)

    def _off(c):
        return pl.multiple_of(base + c * _CB, _CB)

    def gather(c, b):
        off = _off(c)
        pltpu.sync_copy(x_hbm.at[pl.ds(off, _CB)], idx_v.at[b])
        for r in range(_CB):
            pltpu.async_copy(
                tab_hbm.at[idx_v.at[b, r, pl.ds(0, 128)]],
                rows_v.at[b, r, pl.ds(0, 128)],
                gsem[b],
            )
            pltpu.async_copy(
                tab_hbm.at[idx_v.at[b, r, pl.ds(128, _H - 128)]],
                rows_v.at[b, r, pl.ds(128, _H - 128)],
                gsem[b],
            )

    def drain_gather(b):
        pltpu.make_async_copy(
            out_hbm.at[pl.ds(0, _CB)], rows_v.at[b], gsem[b]
        ).wait()

    def writeout(c, b):
        pltpu.async_copy(rows_v.at[b], out_hbm.at[pl.ds(_off(c), _CB)], wsem[b])

    def drain_writeout(b):
        pltpu.make_async_copy(
            rows_v.at[b], out_hbm.at[pl.ds(0, _CB)], wsem[b]
        ).wait()

    gather(0, 0)
    drain_gather(0)
    writeout(0, 0)
    gather(1, 1)

    @pl.loop(1, _NSTEP // 2)
    def _(g):
        c = pl.multiple_of(2 * g, 2)
        drain_gather(1)
        writeout(c - 1, 1)
        drain_writeout(0)
        gather(c, 0)
        drain_gather(0)
        writeout(c, 0)
        drain_writeout(1)
        gather(c + 1, 1)

    drain_gather(1)
    writeout(_NSTEP - 1, 1)
    drain_writeout(0)
    drain_writeout(1)


@jax.jit
def kernel(x, embed):
    run = functools.partial(
        pl.kernel,
        mesh=plsc.VectorSubcoreMesh(core_axis_name="c", subcore_axis_name="s"),
        out_type=jax.ShapeDtypeStruct((_B, _H, _D), jnp.float32),
        scratch_types=[
            pltpu.VMEM((2, _CB, _H), jnp.int32),
            pltpu.VMEM((2, _CB, _H, _D), jnp.float32),
            pltpu.SemaphoreType.DMA,
            pltpu.SemaphoreType.DMA,
            pltpu.SemaphoreType.DMA,
            pltpu.SemaphoreType.DMA,
        ],
        compiler_params=pltpu.CompilerParams(use_tc_tiling_on_sc=False),
    )(_body)
    return run(x, embed)


# speedup vs baseline: 1.4934x; 1.0019x over previous
import functools

import jax
import jax.numpy as jnp
from jax import lax
from jax.experimental import pallas as pl
from jax.experimental.pallas import tpu as pltpu
from jax.experimental.pallas import tpu_sc as plsc

_B = 4096
_H = 200
_D = 32
_NW = 32
_ROWS_W = _B // _NW
_CB = 8
_NSTEP = _ROWS_W // _CB


def _body(x_hbm, tab_hbm, out_hbm, idx_v, rows_v, gsem0, gsem1, wsem0, wsem1):
    wid = lax.axis_index("s") * 2 + lax.axis_index("c")
    base = wid * _ROWS_W
    gsem = (gsem0, gsem1)
    wsem = (wsem0, wsem1)

    def _off(c):
        return pl.multiple_of(base + c * _CB, _CB)

    def gather(c, b):
        loc = pl.multiple_of(c * _CB, _CB)
        for r in range(_CB):
            pltpu.async_copy(
                tab_hbm.at[idx_v.at[loc + r, pl.ds(0, 128)]],
                rows_v.at[b, r, pl.ds(0, 128)],
                gsem[b],
            )
            pltpu.async_copy(
                tab_hbm.at[idx_v.at[loc + r, pl.ds(128, _H - 128)]],
                rows_v.at[b, r, pl.ds(128, _H - 128)],
                gsem[b],
            )

    def drain_gather(b):
        pltpu.make_async_copy(
            out_hbm.at[pl.ds(0, _CB)], rows_v.at[b], gsem[b]
        ).wait()

    def writeout(c, b):
        pltpu.async_copy(rows_v.at[b], out_hbm.at[pl.ds(_off(c), _CB)], wsem[b])

    def drain_writeout(b):
        pltpu.make_async_copy(
            rows_v.at[b], out_hbm.at[pl.ds(0, _CB)], wsem[b]
        ).wait()

    pltpu.sync_copy(x_hbm.at[pl.ds(base, _ROWS_W)], idx_v)

    gather(0, 0)
    drain_gather(0)
    writeout(0, 0)
    gather(1, 1)

    @pl.loop(1, _NSTEP // 2)
    def _(g):
        c = pl.multiple_of(2 * g, 2)
        drain_gather(1)
        writeout(c - 1, 1)
        drain_writeout(0)
        gather(c, 0)
        drain_gather(0)
        writeout(c, 0)
        drain_writeout(1)
        gather(c + 1, 1)

    drain_gather(1)
    writeout(_NSTEP - 1, 1)
    drain_writeout(0)
    drain_writeout(1)


@jax.jit
def kernel(x, embed):
    run = functools.partial(
        pl.kernel,
        mesh=plsc.VectorSubcoreMesh(core_axis_name="c", subcore_axis_name="s"),
        out_type=jax.ShapeDtypeStruct((_B, _H, _D), jnp.float32),
        scratch_types=[
            pltpu.VMEM((_ROWS_W, _H), jnp.int32),
            pltpu.VMEM((2, _CB, _H, _D), jnp.float32),
            pltpu.SemaphoreType.DMA,
            pltpu.SemaphoreType.DMA,
            pltpu.SemaphoreType.DMA,
            pltpu.SemaphoreType.DMA,
        ],
        compiler_params=pltpu.CompilerParams(use_tc_tiling_on_sc=False),
    )(_body)
    return run(x, embed)
